# baseline (speedup 1.0000x reference)
import functools

import jax
import jax.numpy as jnp
from jax import lax
from jax.experimental import pallas as pl
from jax.experimental.pallas import tpu as pltpu
from jax.experimental.pallas import tpu_sc as plsc

DEG = 16
D = 256
NC = 2
NS = 16
NW = NC * NS
SB = 8
HALF = SB // 2
IDXB = 72
IDXC = 2 * IDXB
LANES = 16
NCH = D // LANES
NBUF = 3


def _sc_gather_pool(idx_flat, table, bias, BP):
    CB = BP // NW
    CHUNKS = CB // SB
    IPT = CHUNKS * IDXC

    mesh = plsc.VectorSubcoreMesh(core_axis_name="c", subcore_axis_name="s")

    @functools.partial(
        pl.kernel,
        mesh=mesh,
        out_type=jax.ShapeDtypeStruct((BP, D), jnp.float32),
        scratch_types=[
            pltpu.VMEM((IPT,), jnp.int32),
            pltpu.VMEM((D,), jnp.float32),
            pltpu.VMEM((2 * NBUF, IDXB, D), jnp.float32),
            pltpu.VMEM((NBUF, SB, D), jnp.float32),
        ] + [pltpu.SemaphoreType.DMA] * (3 * NBUF),
    )
    def sc_kernel(idx_hbm, tab_hbm, b_hbm, out_hbm,
                  idx_v, b_v, nb_v, o_v, *sems):
        sem_g = sems[:2 * NBUF]
        sem_w = sems[2 * NBUF:]
        wid = lax.axis_index("s") * NC + lax.axis_index("c")
        base = wid * CB
        pltpu.sync_copy(idx_hbm.at[pl.ds(wid * IPT, IPT)], idx_v)
        pltpu.sync_copy(b_hbm, b_v)

        def gathers(g, b):
            i0 = g * IDXC
            return [
                pltpu.make_async_copy(
                    tab_hbm.at[idx_v.at[pl.ds(i0 + h * IDXB, IDXB)]],
                    nb_v.at[2 * b + h], sem_g[2 * b + h])
                for h in range(2)
            ]

        def out_write(g, b):
            return pltpu.make_async_copy(
                o_v.at[b], out_hbm.at[pl.ds(base + g * SB, SB)], sem_w[b])

        def start_gathers(g, b):
            for c in gathers(g, b):
                c.start()

        def wait_gathers(g, b):
            for c in gathers(g, b):
                c.wait()

        def do_chunk(g, b, wait_write):
            if wait_write:
                @pl.when(g >= NBUF)
                def _():
                    out_write(g - NBUF, b).wait()
            wait_gathers(g, b)

            def accum_i(i, c2, b=b):
                bufi = 2 * b + i // HALF
                r0 = (i % HALF) * (DEG + 1)
                for c in range(NCH):
                    col = c * LANES
                    s = nb_v[bufi, r0, pl.ds(col, LANES)]
                    for j in range(1, DEG + 1):
                        s = s + nb_v[bufi, r0 + j, pl.ds(col, LANES)]
                    s = s + b_v[pl.ds(col, LANES)]
                    o_v[b, i, pl.ds(col, LANES)] = jnp.maximum(s, 0.0)
                return c2

            lax.fori_loop(0, SB, accum_i, 0)
            out_write(g, b).start()

        start_gathers(0, 0)
        start_gathers(1, 1)

        KMAIN = (CHUNKS // NBUF) * NBUF

        def body(k, carry):
            for b in range(NBUF):
                g = k * NBUF + b
                nxt = g + 2
                bn = (b + 2) % NBUF

                @pl.when(nxt < CHUNKS)
                def _(nxt=nxt, bn=bn):
                    start_gathers(nxt, bn)

                do_chunk(g, b, wait_write=True)
            return carry

        lax.fori_loop(0, KMAIN // NBUF, body, 0)

        for g in range(KMAIN, CHUNKS):
            b = g % NBUF
            out_write(g - NBUF, b).wait()
            do_chunk(g, b, wait_write=False)

        for t in range(CHUNKS - NBUF, CHUNKS):
            out_write(t, t % NBUF).wait()

    return sc_kernel(idx_flat, table, bias)


def _tab_body(feat_ref, w_ref, o_ref):
    o_ref[...] = jnp.dot(feat_ref[...], w_ref[0],
                         preferred_element_type=jnp.float32)


def _tc_tables(features, W_stk, N, BM=1000):
    nb = N // BM
    return pl.pallas_call(
        _tab_body,
        grid=(2, nb),
        in_specs=[
            pl.BlockSpec((BM, D), lambda j, i: (i, 0)),
            pl.BlockSpec((1, D, D), lambda j, i: (j, 0, 0)),
        ],
        out_specs=pl.BlockSpec((BM, D), lambda j, i: (j * nb + i, 0)),
        out_shape=jax.ShapeDtypeStruct((2 * N, D), jnp.float32),
    )(features, W_stk)


@jax.jit
def kernel(nodes, neighbors, features, W, b):
    B = nodes.shape[0]
    N = features.shape[0]
    step = NW * SB
    BP = ((B + step - 1) // step) * step
    pad = BP - B
    nodes_p = jnp.pad(nodes.astype(jnp.int32), (0, pad))
    nbr_p = jnp.pad(neighbors.astype(jnp.int32), ((0, pad), (0, 0)))

    aug = jnp.concatenate([nodes_p[:, None], nbr_p + N], axis=1)
    halves = aug.reshape(BP // HALF, HALF * (DEG + 1))
    idx_flat = jnp.pad(halves, ((0, 0), (0, IDXB - HALF * (DEG + 1)))).reshape(-1)

    W_stk = jnp.stack([W[:D], W[D:] * (1.0 / DEG)])
    table = _tc_tables(features, W_stk, N)

    out_p = _sc_gather_pool(idx_flat, table, b, BP)
    return out_p[:B]


# speedup vs baseline: 1.6715x; 1.6715x over previous
import functools

import jax
import jax.numpy as jnp
from jax import lax
from jax.experimental import pallas as pl
from jax.experimental.pallas import tpu as pltpu
from jax.experimental.pallas import tpu_sc as plsc

DEG = 16
D = 256
NC = 2
NS = 16
NW = NC * NS
SB = 8
HALF = SB // 2
IDXB = 72
IDXC = 2 * IDXB
LANES = 16
NCH = D // LANES
NBUF = 3


def _sc_gather_pool(idx_flat, table, bias, BP):
    CB = BP // NW
    CHUNKS = CB // SB
    IPT = CHUNKS * IDXC

    mesh = plsc.VectorSubcoreMesh(core_axis_name="c", subcore_axis_name="s")

    @functools.partial(
        pl.kernel,
        mesh=mesh,
        out_type=jax.ShapeDtypeStruct((BP, D), jnp.float32),
        scratch_types=[
            pltpu.VMEM((IPT,), jnp.int32),
            pltpu.VMEM((D,), jnp.float32),
            pltpu.VMEM((2 * NBUF, IDXB, D), jnp.float32),
            pltpu.VMEM((NBUF, SB, D), jnp.float32),
        ] + [pltpu.SemaphoreType.DMA] * (3 * NBUF),
    )
    def sc_kernel(idx_hbm, tab_hbm, b_hbm, out_hbm,
                  idx_v, b_v, nb_v, o_v, *sems):
        sem_g = sems[:2 * NBUF]
        sem_w = sems[2 * NBUF:]
        wid = lax.axis_index("s") * NC + lax.axis_index("c")
        base = wid * CB
        pltpu.sync_copy(idx_hbm.at[pl.ds(wid * IPT, IPT)], idx_v)
        pltpu.sync_copy(b_hbm, b_v)

        def gathers(g, b):
            i0 = g * IDXC
            return [
                pltpu.make_async_copy(
                    tab_hbm.at[idx_v.at[pl.ds(i0 + h * IDXB, IDXB)]],
                    nb_v.at[2 * b + h], sem_g[2 * b + h])
                for h in range(2)
            ]

        def out_write(g, b):
            return pltpu.make_async_copy(
                o_v.at[b], out_hbm.at[pl.ds(base + g * SB, SB)], sem_w[b])

        def start_gathers(g, b):
            for c in gathers(g, b):
                c.start()

        def wait_gathers(g, b):
            for c in gathers(g, b):
                c.wait()

        def do_chunk(g, b, wait_write):
            if wait_write:
                @pl.when(g >= NBUF)
                def _():
                    out_write(g - NBUF, b).wait()
            wait_gathers(g, b)

            def accum_i(i, c2, b=b):
                bufi = 2 * b + i // HALF
                r0 = (i % HALF) * (DEG + 1)
                for c in range(NCH):
                    col = c * LANES
                    s = nb_v[bufi, r0, pl.ds(col, LANES)]
                    for j in range(1, DEG + 1):
                        s = s + nb_v[bufi, r0 + j, pl.ds(col, LANES)]
                    s = s + b_v[pl.ds(col, LANES)]
                    o_v[b, i, pl.ds(col, LANES)] = jnp.maximum(s, 0.0)
                return c2

            lax.fori_loop(0, SB, accum_i, 0)
            out_write(g, b).start()

        start_gathers(0, 0)
        start_gathers(1, 1)

        KMAIN = (CHUNKS // NBUF) * NBUF

        def body(k, carry):
            for b in range(NBUF):
                g = k * NBUF + b
                nxt = g + 2
                bn = (b + 2) % NBUF

                @pl.when(nxt < CHUNKS)
                def _(nxt=nxt, bn=bn):
                    start_gathers(nxt, bn)

                do_chunk(g, b, wait_write=True)
            return carry

        lax.fori_loop(0, KMAIN // NBUF, body, 0)

        for g in range(KMAIN, CHUNKS):
            b = g % NBUF
            out_write(g - NBUF, b).wait()
            do_chunk(g, b, wait_write=False)

        for t in range(CHUNKS - NBUF, CHUNKS):
            out_write(t, t % NBUF).wait()

    return sc_kernel(idx_flat, table, bias)


def _tab_body(feat_ref, w_ref, o_ref):
    o_ref[...] = jnp.dot(feat_ref[...], w_ref[0],
                         preferred_element_type=jnp.float32)


def _tc_tables(features, W_stk, N, BM=1000):
    nb = N // BM
    return pl.pallas_call(
        _tab_body,
        grid=(2, nb),
        in_specs=[
            pl.BlockSpec((BM, D), lambda j, i: (i, 0)),
            pl.BlockSpec((1, D, D), lambda j, i: (j, 0, 0)),
        ],
        out_specs=pl.BlockSpec((BM, D), lambda j, i: (j * nb + i, 0)),
        out_shape=jax.ShapeDtypeStruct((2 * N, D), jnp.float32),
    )(features, W_stk)


@jax.jit
def kernel(nodes, neighbors, features, W, b):
    B = nodes.shape[0]
    N = features.shape[0]
    step = NW * SB
    BP = ((B + step - 1) // step) * step
    pad = BP - B
    nodes_p = jnp.pad(nodes.astype(jnp.int32), (0, pad))
    nbr_p = jnp.pad(neighbors.astype(jnp.int32), ((0, pad), (0, 0)))

    aug = jnp.concatenate([nodes_p[:, None], nbr_p + N], axis=1)
    halves = aug.reshape(BP // HALF, HALF * (DEG + 1))
    npadi = IDXB - HALF * (DEG + 1)
    padi = (jnp.arange(BP // HALF * npadi, dtype=jnp.int32) * 997) % (2 * N)
    idx_flat = jnp.concatenate(
        [halves, padi.reshape(BP // HALF, npadi)], axis=1).reshape(-1)

    W_stk = jnp.stack([W[:D], W[D:] * (1.0 / DEG)])
    table = _tc_tables(features, W_stk, N)

    out_p = _sc_gather_pool(idx_flat, table, b, BP)
    return out_p[:B]


# speedup vs baseline: 4.4919x; 2.6873x over previous
import functools

import jax
import jax.numpy as jnp
from jax import lax
from jax.experimental import pallas as pl
from jax.experimental.pallas import tpu as pltpu
from jax.experimental.pallas import tpu_sc as plsc

DEG = 16
D = 256
DP = D // 2
NC = 2
NS = 16
NW = NC * NS
SB = 8
LANES = 16
NGRP = DP // LANES
NBUF = 3


def _sc_gather_pool(idx_nbr, idx_self, table, bias, BP):
    CB = BP // NW
    CHUNKS = CB // SB

    mesh = plsc.VectorSubcoreMesh(core_axis_name="c", subcore_axis_name="s")

    @functools.partial(
        pl.kernel,
        mesh=mesh,
        out_type=jax.ShapeDtypeStruct((BP, D), jnp.float32),
        scratch_types=[
            pltpu.VMEM((CB * DEG,), jnp.int32),
            pltpu.VMEM((CB,), jnp.int32),
            pltpu.VMEM((D,), jnp.float32),
            pltpu.VMEM((NBUF, SB * DEG, DP), jnp.int32),
            pltpu.VMEM((NBUF, SB, DP), jnp.int32),
            pltpu.VMEM((NBUF, SB, D), jnp.float32),
        ] + [pltpu.SemaphoreType.DMA] * (3 * NBUF),
    )
    def sc_kernel(idxn_hbm, idxs_hbm, tab_hbm, b_hbm, out_hbm,
                  idxn_v, idxs_v, b_v, nb_v, sf_v, o_v, *sems):
        sem_n = sems[:NBUF]
        sem_s = sems[NBUF:2 * NBUF]
        sem_w = sems[2 * NBUF:]
        wid = lax.axis_index("s") * NC + lax.axis_index("c")
        base = wid * CB
        pltpu.sync_copy(idxn_hbm.at[pl.ds(base * DEG, CB * DEG)], idxn_v)
        pltpu.sync_copy(idxs_hbm.at[pl.ds(base, CB)], idxs_v)
        pltpu.sync_copy(b_hbm, b_v)

        def gathers(g, b):
            return [
                pltpu.make_async_copy(
                    tab_hbm.at[idxn_v.at[pl.ds(g * SB * DEG, SB * DEG)]],
                    nb_v.at[b], sem_n[b]),
                pltpu.make_async_copy(
                    tab_hbm.at[idxs_v.at[pl.ds(g * SB, SB)]],
                    sf_v.at[b], sem_s[b]),
            ]

        def out_write(g, b):
            return pltpu.make_async_copy(
                o_v.at[b], out_hbm.at[pl.ds(base + g * SB, SB)], sem_w[b])

        def start_gathers(g, b):
            for c in gathers(g, b):
                c.start()

        def wait_gathers(g, b):
            for c in gathers(g, b):
                c.wait()

        def do_chunk(g, b, wait_write):
            if wait_write:
                @pl.when(g >= NBUF)
                def _():
                    out_write(g - NBUF, b).wait()
            wait_gathers(g, b)

            def accum_i(i, c2, b=b):
                r0 = i * DEG
                mask = jnp.full((LANES,), -65536, jnp.int32)
                sh16 = jnp.full((LANES,), 16, jnp.int32)
                for m in range(NGRP):
                    col = m * LANES
                    x = sf_v[b, i, pl.ds(col, LANES)]
                    se = lax.bitcast_convert_type(x << sh16, jnp.float32)
                    so = lax.bitcast_convert_type(x & mask, jnp.float32)
                    for j in range(DEG):
                        x = nb_v[b, r0 + j, pl.ds(col, LANES)]
                        se = se + lax.bitcast_convert_type(x << sh16, jnp.float32)
                        so = so + lax.bitcast_convert_type(x & mask, jnp.float32)
                    se = se + b_v[pl.ds(2 * col, LANES)]
                    so = so + b_v[pl.ds(2 * col + LANES, LANES)]
                    o_v[b, i, pl.ds(2 * col, LANES)] = jnp.maximum(se, 0.0)
                    o_v[b, i, pl.ds(2 * col + LANES, LANES)] = jnp.maximum(so, 0.0)
                return c2

            lax.fori_loop(0, SB, accum_i, 0)
            out_write(g, b).start()

        start_gathers(0, 0)
        start_gathers(1, 1)

        KMAIN = (CHUNKS // NBUF) * NBUF

        def body(k, carry):
            for b in range(NBUF):
                g = k * NBUF + b
                nxt = g + 2
                bn = (b + 2) % NBUF

                @pl.when(nxt < CHUNKS)
                def _(nxt=nxt, bn=bn):
                    start_gathers(nxt, bn)

                do_chunk(g, b, wait_write=True)
            return carry

        lax.fori_loop(0, KMAIN // NBUF, body, 0)

        for g in range(KMAIN, CHUNKS):
            b = g % NBUF
            out_write(g - NBUF, b).wait()
            do_chunk(g, b, wait_write=False)

        for t in range(CHUNKS - NBUF, CHUNKS):
            out_write(t, t % NBUF).wait()

    return sc_kernel(idx_nbr, idx_self, table, bias)


def _tab_body(feat_ref, w_ref, o_ref):
    acc = jnp.dot(feat_ref[...], w_ref[0],
                  preferred_element_type=jnp.float32)
    lo = pltpu.bitcast(acc[:, :DP].astype(jnp.bfloat16),
                       jnp.uint16).astype(jnp.uint32)
    hi = pltpu.bitcast(acc[:, DP:].astype(jnp.bfloat16),
                       jnp.uint16).astype(jnp.uint32)
    o_ref[...] = pltpu.bitcast((hi << 16) | lo, jnp.int32)


def _tc_tables(features, W_stk, N, BM=1000):
    nb = N // BM
    return pl.pallas_call(
        _tab_body,
        grid=(2, nb),
        in_specs=[
            pl.BlockSpec((BM, D), lambda j, i: (i, 0)),
            pl.BlockSpec((1, D, D), lambda j, i: (j, 0, 0)),
        ],
        out_specs=pl.BlockSpec((BM, DP), lambda j, i: (j * nb + i, 0)),
        out_shape=jax.ShapeDtypeStruct((2 * N, DP), jnp.int32),
    )(features, W_stk)


@jax.jit
def kernel(nodes, neighbors, features, W, b):
    B = nodes.shape[0]
    N = features.shape[0]
    step = NW * SB
    BP = ((B + step - 1) // step) * step
    pad = BP - B
    dummy_n = (jnp.arange(pad * DEG, dtype=jnp.int32) * 997) % N
    idx_nbr = jnp.concatenate(
        [neighbors.astype(jnp.int32).reshape(-1), dummy_n])
    dummy_s = (jnp.arange(pad, dtype=jnp.int32) * 997) % N
    idx_self = jnp.concatenate(
        [nodes.astype(jnp.int32) + N, dummy_s + N])

    p_ = jnp.arange(DP)
    idx_lo = (p_ // LANES) * 32 + (p_ % LANES)
    col_perm = jnp.concatenate([idx_lo, idx_lo + LANES])

    W_stk = jnp.stack([W[D:] * (1.0 / DEG), W[:D]])[:, :, col_perm]
    table_i32 = _tc_tables(features, W_stk, N)

    out_p = _sc_gather_pool(idx_nbr, idx_self, table_i32, b, BP)
    return out_p[:B]
